# Initial kernel scaffold; baseline (speedup 1.0000x reference)
import jax
import jax.numpy as jnp
from jax.experimental import pallas as pl


def kernel(inputs, W_enc1, b_enc1, W_enc2, b_enc2, rotations, W_edge1, b_edge1, W_edge2, b_edge2):
    raise NotImplementedError("write your pallas kernel here")


# speedup vs baseline: 10.6562x; 10.6562x over previous
import jax
import jax.numpy as jnp
from jax import lax
from jax.experimental import pallas as pl

B = 2
N = 4000
D_IN = 12
DIST_DIM = 128
HIDDEN = 128
MAX_BINS = 200
BIN_SIZE = 500
NBINS = N // BIN_SIZE
K = 5
CHUNK = 500
NCHUNK = N // CHUNK

_BF = jnp.bfloat16
_F32 = jnp.float32


def _elu(x):
    return jnp.where(x > 0, x, jnp.exp(x) - 1.0)


def _sigmoid(x):
    return 1.0 / (1.0 + jnp.exp(-x))


def _bdot(a, b, dn=None):
    ab = a.astype(_BF)
    bb = b.astype(_BF)
    if dn is None:
        return jnp.dot(ab, bb, preferred_element_type=_F32)
    return lax.dot_general(ab, bb, dn, preferred_element_type=_F32)


def _enc1_kernel(x_ref, w1_ref, b1_ref, pre_ref):
    pre_ref[0] = _bdot(x_ref[0], w1_ref[...]) + b1_ref[...]


def _encode_kernel(h_ref, w2_ref, b2_ref, rot_ref, emb_ref, pos_ref):
    emb = _bdot(h_ref[0], w2_ref[...]) + b2_ref[...]
    embb = emb.astype(_BF)
    emb_ref[0] = embb

    proj = _bdot(embb, rot_ref[...])
    lane = lax.broadcasted_iota(jnp.int32, (N, 128), 1)
    neg = jnp.float32(-3.0e38)
    pm = jnp.where(lane < MAX_BINS // 2, proj, neg)
    nm = jnp.where(lane < MAX_BINS // 2, -proj, neg)
    cmul = jnp.concatenate([pm, nm], axis=1)
    lane2 = lax.broadcasted_iota(jnp.int32, (N, 256), 1)
    mx = jnp.max(cmul, axis=1, keepdims=True)
    eq = cmul == mx
    binc = jnp.min(jnp.where(eq, lane2, 256), axis=1, keepdims=True)
    onehot = (lane2 == binc).astype(_F32)

    hist = jnp.sum(onehot, axis=0, keepdims=True)
    c0 = lax.broadcasted_iota(jnp.int32, (256, 256), 0)
    c1i = lax.broadcasted_iota(jnp.int32, (256, 256), 1)
    lt256 = (c0 < c1i).astype(_F32)
    csum_excl = jnp.dot(hist, lt256, precision=lax.Precision.HIGHEST,
                        preferred_element_type=_F32)
    r0 = lax.broadcasted_iota(jnp.int32, (CHUNK, CHUNK), 0)
    r1 = lax.broadcasted_iota(jnp.int32, (CHUNK, CHUNK), 1)
    ls = (r1 < r0).astype(_F32)
    prev = jnp.zeros((1, 256), _F32)
    chunks = []
    for g in range(NCHUNK):
        oh = onehot[g * CHUNK:(g + 1) * CHUNK]
        cum2 = _bdot(ls, oh) + prev
        chunks.append(jnp.sum(oh * (cum2 + csum_excl), axis=1, keepdims=True))
        prev = prev + jnp.sum(oh, axis=0, keepdims=True)
    pos_ref[0] = jnp.concatenate(chunks, axis=0).astype(jnp.int32)


def _bin_kernel(posr_ref, emb_ref, inp_ref, a16_ref, b16_ref, c1_ref,
                be1_ref, w2_ref, be2_ref, out_ref):
    n = pl.program_id(1)
    posr = posr_ref[...].reshape(1, N)
    rsub = lax.broadcasted_iota(jnp.int32, (BIN_SIZE, CHUNK), 0)
    sub = jnp.zeros((BIN_SIZE, DIST_DIM), _F32)
    sinp = jnp.zeros((BIN_SIZE, 16), _F32)
    base = n * BIN_SIZE
    for g in range(NCHUNK):
        pr = posr[:, g * CHUNK:(g + 1) * CHUNK]
        pb = (pr == rsub + base).astype(_BF)
        sub = sub + jnp.dot(pb, emb_ref[0, g * CHUNK:(g + 1) * CHUNK, :],
                            preferred_element_type=_F32)
        sinp = sinp + _bdot(pb, inp_ref[0, g * CHUNK:(g + 1) * CHUNK, :])

    dm = _sigmoid(_bdot(sub, sub, (((1,), (1,)), ((), ()))))

    coli = lax.broadcasted_iota(jnp.int32, (BIN_SIZE, BIN_SIZE), 1)
    x2s, ms = [], []
    dmw = dm
    for _ in range(K):
        mk = jnp.max(dmw, axis=1, keepdims=True)
        idxk = jnp.min(jnp.where(dmw == mk, coli, BIN_SIZE),
                       axis=1, keepdims=True)
        oh = (coli == idxk).astype(_BF)
        x2s.append(_bdot(oh, sinp))
        ms.append(mk)
        dmw = jnp.where(coli == idxk, -1.0, dmw)

    x1 = jnp.concatenate([sinp] * K, axis=0)
    x2 = jnp.concatenate(x2s, axis=0)
    mv = jnp.concatenate(ms, axis=0)
    mvb = mv.astype(_BF).astype(_F32)
    c1b = c1_ref[...].astype(_BF).astype(_F32)
    hh = _elu(_bdot(x1, a16_ref[...]) + _bdot(x2, b16_ref[...])
              + mvb * c1b + be1_ref[...])
    o = _bdot(hh, w2_ref[...])[:, 0:1] + be2_ref[...]
    out_ref[0, 0] = _sigmoid(o)


def _stage1(inp16, w1p, b1r, W_enc2, b2r, rotp):
    pre = pl.pallas_call(
        _enc1_kernel,
        grid=(B,),
        in_specs=[
            pl.BlockSpec((1, N, 16), lambda b: (b, 0, 0)),
            pl.BlockSpec((16, HIDDEN), lambda b: (0, 0)),
            pl.BlockSpec((1, HIDDEN), lambda b: (0, 0)),
        ],
        out_specs=pl.BlockSpec((1, N, HIDDEN), lambda b: (b, 0, 0)),
        out_shape=jax.ShapeDtypeStruct((B, N, HIDDEN), _F32),
    )(inp16, w1p, b1r)
    h = jnp.where(pre > 0, pre, jnp.expm1(pre))
    return pl.pallas_call(
        _encode_kernel,
        grid=(B,),
        in_specs=[
            pl.BlockSpec((1, N, HIDDEN), lambda b: (b, 0, 0)),
            pl.BlockSpec((HIDDEN, DIST_DIM), lambda b: (0, 0)),
            pl.BlockSpec((1, DIST_DIM), lambda b: (0, 0)),
            pl.BlockSpec((DIST_DIM, 128), lambda b: (0, 0)),
        ],
        out_specs=[
            pl.BlockSpec((1, N, DIST_DIM), lambda b: (b, 0, 0)),
            pl.BlockSpec((1, N, 1), lambda b: (b, 0, 0)),
        ],
        out_shape=[
            jax.ShapeDtypeStruct((B, N, DIST_DIM), _BF),
            jax.ShapeDtypeStruct((B, N, 1), jnp.int32),
        ],
    )(h, W_enc2, b2r, rotp)


def _stage2(posr, emb, inp16, a16, b16, c1, be1, w2p, be2):
    return pl.pallas_call(
        _bin_kernel,
        grid=(B, NBINS),
        in_specs=[
            pl.BlockSpec((1, 1, N), lambda b, n: (b, 0, 0)),
            pl.BlockSpec((1, N, DIST_DIM), lambda b, n: (b, 0, 0)),
            pl.BlockSpec((1, N, 16), lambda b, n: (b, 0, 0)),
            pl.BlockSpec((16, HIDDEN), lambda b, n: (0, 0)),
            pl.BlockSpec((16, HIDDEN), lambda b, n: (0, 0)),
            pl.BlockSpec((1, HIDDEN), lambda b, n: (0, 0)),
            pl.BlockSpec((1, HIDDEN), lambda b, n: (0, 0)),
            pl.BlockSpec((HIDDEN, 8), lambda b, n: (0, 0)),
            pl.BlockSpec((1, 1), lambda b, n: (0, 0)),
        ],
        out_specs=pl.BlockSpec((1, 1, K * BIN_SIZE, 1),
                               lambda b, n: (b, n, 0, 0)),
        out_shape=jax.ShapeDtypeStruct((B, NBINS, K * BIN_SIZE, 1), _F32),
    )(posr, emb, inp16, a16, b16, c1, be1, w2p, be2)


def kernel(inputs, W_enc1, b_enc1, W_enc2, b_enc2, rotations,
           W_edge1, b_edge1, W_edge2, b_edge2):
    inp16 = jnp.pad(inputs, ((0, 0), (0, 0), (0, 16 - D_IN)))
    w1p = jnp.pad(W_enc1, ((0, 4), (0, 0)))
    b1r = b_enc1.reshape(1, HIDDEN)
    b2r = b_enc2.reshape(1, DIST_DIM)
    rotp = jnp.pad(rotations, ((0, 0), (0, 128 - MAX_BINS // 2)))
    a16 = jnp.pad(W_edge1[0:D_IN], ((0, 4), (0, 0)))
    b16 = jnp.pad(W_edge1[D_IN:2 * D_IN], ((0, 4), (0, 0)))
    c1 = W_edge1[2 * D_IN:2 * D_IN + 1]
    be1 = b_edge1.reshape(1, HIDDEN)
    w2p = jnp.pad(W_edge2, ((0, 0), (0, 7)))
    be2 = b_edge2.reshape(1, 1)

    emb, pos = _stage1(inp16, w1p, b1r, W_enc2, b2r, rotp)
    posr = pos.reshape(B, 1, N)
    out = _stage2(posr, emb, inp16, a16, b16, c1, be1, w2p, be2)
    return (out.reshape(B, NBINS, K, BIN_SIZE)
            .transpose(0, 1, 3, 2).reshape(B, N, K))
